# Initial kernel scaffold; baseline (speedup 1.0000x reference)
import jax
import jax.numpy as jnp
from jax.experimental import pallas as pl


def kernel(input_ids, word_emb, pos_emb, gamma, beta):
    raise NotImplementedError("write your pallas kernel here")


# speedup vs baseline: 2.1386x; 2.1386x over previous
import functools

import jax
import jax.numpy as jnp
from jax import lax
from jax.experimental import pallas as pl
from jax.experimental.pallas import tpu as pltpu
from jax.experimental.pallas import tpu_sc as plsc

NC = 2
NS = 16
NW = NC * NS
LANES = 16

DIM = 128
KV = DIM // LANES
EPS = 1e-12


def _lane_sum(v):
    lanes = lax.iota(jnp.int32, LANES)
    for s in (8, 4, 2, 1):
        idx = jnp.bitwise_xor(lanes, s)
        v = v + jnp.take(v, idx, axis=0, mode="promise_in_bounds")
    return v


def _rsqrt(x):
    i = lax.bitcast_convert_type(x, jnp.int32)
    i = jnp.int32(0x5F3759DF) - lax.shift_right_logical(i, 1)
    y = lax.bitcast_convert_type(i, jnp.float32)
    for _ in range(3):
        y = y * (1.5 - 0.5 * x * y * y)
    return y


def _make_sc_kernel(n_chunks, chunk, seqlen):
    rows_per_w = n_chunks * chunk
    total = NW * rows_per_w
    mesh = plsc.VectorSubcoreMesh(
        core_axis_name="c", subcore_axis_name="s", num_cores=NC, num_subcores=NS
    )

    @functools.partial(
        pl.kernel,
        mesh=mesh,
        out_type=jax.ShapeDtypeStruct((total, DIM), jnp.float32),
        scratch_types=[
            pltpu.VMEM((n_chunks, chunk), jnp.int32),
            pltpu.VMEM((seqlen, DIM), jnp.float32),
            pltpu.VMEM((chunk, DIM), jnp.float32),
            pltpu.VMEM((DIM,), jnp.float32),
            pltpu.VMEM((DIM,), jnp.float32),
            pltpu.SemaphoreType.DMA,
        ],
        compiler_params=pltpu.CompilerParams(needs_layout_passes=False),
    )
    def body(ids_hbm, word_hbm, pos_hbm, g_hbm, b_hbm, out_hbm,
             idx_v, pos_v, rows_v, g_v, b_v, sem):
        wid = lax.axis_index("s") * NC + lax.axis_index("c")
        pltpu.sync_copy(ids_hbm.at[wid], idx_v)
        pltpu.sync_copy(pos_hbm, pos_v)
        pltpu.sync_copy(g_hbm, g_v)
        pltpu.sync_copy(b_hbm, b_v)

        gs = [g_v[pl.ds(k * LANES, LANES)] for k in range(KV)]
        bs = [b_v[pl.ds(k * LANES, LANES)] for k in range(KV)]
        base_w = wid * rows_per_w

        def chunk_body(c, _):
            pltpu.async_copy(word_hbm.at[idx_v.at[c]], rows_v, sem).wait()

            def row_body(r, _):
                p = lax.rem(c * chunk + r, seqlen)
                xs = []
                for k in range(KV):
                    sl = pl.ds(k * LANES, LANES)
                    xs.append(rows_v[r, sl] + pos_v[p, sl])
                acc = xs[0]
                acc2 = xs[0] * xs[0]
                for k in range(1, KV):
                    acc = acc + xs[k]
                    acc2 = acc2 + xs[k] * xs[k]
                s1 = jnp.sum(acc)
                s2 = jnp.sum(acc2)
                mu = jnp.broadcast_to(s1, (LANES,)) * (1.0 / DIM)
                ex2 = jnp.broadcast_to(s2, (LANES,)) * (1.0 / DIM)
                inv = _rsqrt(ex2 - mu * mu + EPS)
                for k in range(KV):
                    rows_v[r, pl.ds(k * LANES, LANES)] = (
                        (xs[k] - mu) * inv * gs[k] + bs[k]
                    )
                return _

            lax.fori_loop(0, chunk, row_body, 0)
            pltpu.sync_copy(rows_v, out_hbm.at[pl.ds(base_w + c * chunk, chunk)])
            return _

        lax.fori_loop(0, n_chunks, chunk_body, 0)

    return body


def kernel(input_ids, word_emb, pos_emb, gamma, beta):
    B, L = input_ids.shape
    D = word_emb.shape[1]
    total = B * L
    chunk = 128
    n_chunks = total // (NW * chunk)
    ids3 = input_ids.reshape(NW, n_chunks, chunk).astype(jnp.int32)
    pos = pos_emb[:L]
    sc = _make_sc_kernel(n_chunks, chunk, L)
    out = sc(ids3, word_emb, pos, gamma, beta)
    return out.reshape(B, L, D)


# speedup vs baseline: 2.7026x; 1.2637x over previous
import functools

import jax
import jax.numpy as jnp
from jax import lax
from jax.experimental import pallas as pl
from jax.experimental.pallas import tpu as pltpu
from jax.experimental.pallas import tpu_sc as plsc

NC = 2
NS = 16
NW = NC * NS
LANES = 16

DIM = 128
KV = DIM // LANES
EPS = 1e-12
CHUNK = 64
NBUF = 4


def _rsqrt(x):
    i = lax.bitcast_convert_type(x, jnp.int32)
    i = jnp.int32(0x5F3759DF) - lax.shift_right_logical(i, 1)
    y = lax.bitcast_convert_type(i, jnp.float32)
    for _ in range(3):
        y = y * (1.5 - 0.5 * x * y * y)
    return y


def _make_sc_kernel(n_chunks, seqlen):
    rows_per_w = n_chunks * CHUNK
    total = NW * rows_per_w
    mesh = plsc.VectorSubcoreMesh(
        core_axis_name="c", subcore_axis_name="s", num_cores=NC, num_subcores=NS
    )

    @functools.partial(
        pl.kernel,
        mesh=mesh,
        out_type=jax.ShapeDtypeStruct((total, DIM), jnp.float32),
        scratch_types=[
            pltpu.VMEM((n_chunks, CHUNK), jnp.int32),
            pltpu.VMEM((seqlen + CHUNK, DIM), jnp.float32),
            [pltpu.VMEM((CHUNK, DIM), jnp.float32) for _ in range(NBUF)],
            pltpu.VMEM((DIM,), jnp.float32),
            pltpu.VMEM((DIM,), jnp.float32),
            [pltpu.SemaphoreType.DMA for _ in range(NBUF)],
            [pltpu.SemaphoreType.DMA for _ in range(NBUF)],
        ],
        compiler_params=pltpu.CompilerParams(needs_layout_passes=False),
    )
    def body(ids_hbm, word_hbm, pos_hbm, g_hbm, b_hbm, out_hbm,
             idx_v, pos_v, bufs, g_v, b_v, gsems, osems):
        wid = lax.axis_index("s") * NC + lax.axis_index("c")
        pltpu.sync_copy(ids_hbm.at[wid], idx_v)
        pltpu.sync_copy(pos_hbm, pos_v)
        pltpu.sync_copy(g_hbm, g_v)
        pltpu.sync_copy(b_hbm, b_v)

        gs = [g_v[pl.ds(k * LANES, LANES)] for k in range(KV)]
        bs = [b_v[pl.ds(k * LANES, LANES)] for k in range(KV)]
        base_w = wid * rows_per_w

        def start_gather(c, b):
            pltpu.async_copy(word_hbm.at[idx_v.at[c]], bufs[b], gsems[b])

        def wait_gather(c, b):
            pltpu.make_async_copy(word_hbm.at[idx_v.at[c]], bufs[b],
                                  gsems[b]).wait()

        def drain_out(b):
            pltpu.make_async_copy(
                bufs[b], out_hbm.at[pl.ds(base_w, CHUNK)], osems[b]
            ).wait()

        def process_row(buf, r, pbase):
            p = pbase + r
            xs = []
            for k in range(KV):
                sl = pl.ds(k * LANES, LANES)
                xs.append(buf[r, sl] + pos_v[p, sl])
            acc = xs[0]
            acc2 = xs[0] * xs[0]
            for k in range(1, KV):
                acc = acc + xs[k]
                acc2 = acc2 + xs[k] * xs[k]
            s1 = jnp.sum(acc)
            s2 = jnp.sum(acc2)
            mu = jnp.broadcast_to(s1, (LANES,)) * (1.0 / DIM)
            ex2 = jnp.broadcast_to(s2, (LANES,)) * (1.0 / DIM)
            inv = _rsqrt(ex2 - mu * mu + EPS)
            for k in range(KV):
                buf[r, pl.ds(k * LANES, LANES)] = (
                    (xs[k] - mu) * inv * gs[k] + bs[k]
                )

        for b in range(NBUF - 1):
            start_gather(b, b)

        def outer(i, _):
            for b in range(NBUF):
                c = i * NBUF + b
                wait_gather(c, b)
                pbase = lax.rem(c * CHUNK, seqlen)
                buf = bufs[b]

                def row_body(j, _):
                    r0 = 2 * j
                    process_row(buf, r0, pbase)
                    process_row(buf, r0 + 1, pbase)
                    return _

                lax.fori_loop(0, CHUNK // 2, row_body, 0)
                pltpu.async_copy(
                    buf, out_hbm.at[pl.ds(base_w + c * CHUNK, CHUNK)], osems[b]
                )
                nb = (b + NBUF - 1) % NBUF
                nc = c + NBUF - 1

                @pl.when(jnp.logical_and(nc < n_chunks, c >= 1))
                def _():
                    drain_out(nb)

                @pl.when(nc < n_chunks)
                def _():
                    start_gather(nc, nb)
            return 0

        lax.fori_loop(0, n_chunks // NBUF, outer, 0)
        for b in range(NBUF):
            drain_out(b)

    return body


def kernel(input_ids, word_emb, pos_emb, gamma, beta):
    B, L = input_ids.shape
    D = word_emb.shape[1]
    total = B * L
    n_chunks = total // (NW * CHUNK)
    ids3 = input_ids.reshape(NW, n_chunks, CHUNK).astype(jnp.int32)
    pos = jnp.concatenate([pos_emb[:L], pos_emb[:CHUNK]], axis=0)
    sc = _make_sc_kernel(n_chunks, L)
    out = sc(ids3, word_emb, pos, gamma, beta)
    return out.reshape(B, L, D)
